# baseline (speedup 1.0000x reference)
import functools

import jax
import jax.numpy as jnp
from jax import lax
from jax.experimental import pallas as pl
from jax.experimental.pallas import tpu as pltpu
from jax.experimental.pallas import tpu_sc as plsc

_B = 16384
_TAU = 5
_NC = 2
_NS = 16
_NW = _NC * _NS
_L = 16
_PER_W = _B // _NW

_mesh = plsc.VectorSubcoreMesh(core_axis_name="c", subcore_axis_name="s")


@functools.partial(
    pl.kernel,
    mesh=_mesh,
    out_type=jax.ShapeDtypeStruct((2, _B), jnp.float32),
    scratch_types=[
        pltpu.VMEM((_PER_W,), jnp.int32),
        pltpu.VMEM((_PER_W,), jnp.float32),
        pltpu.VMEM((_PER_W,), jnp.float32),
    ],
    compiler_params=pltpu.CompilerParams(
        needs_layout_passes=False,
        skip_device_barrier=True,
        disable_bounds_checks=True,
        disable_semaphore_checks=True,
    ),
)
def _router_sc(it_hbm, out_hbm, it_v, c0_v, c1_v):
    wid = lax.axis_index("s") * _NC + lax.axis_index("c")
    base = wid * _PER_W
    pltpu.sync_copy(it_hbm.at[pl.ds(base, _PER_W)], it_v)

    def body(i, carry):
        sl = pl.ds(i * _L, _L)
        x = it_v[sl]
        hit = lax.rem(x + 1, _TAU) == 0
        col1 = jnp.where(hit, jnp.float32(1.0), jnp.float32(0.0))
        c1_v[sl] = col1
        c0_v[sl] = jnp.float32(1.0) - col1
        return carry

    lax.fori_loop(0, _PER_W // _L, body, 0)
    pltpu.sync_copy(c0_v, out_hbm.at[0, pl.ds(base, _PER_W)])
    pltpu.sync_copy(c1_v, out_hbm.at[1, pl.ds(base, _PER_W)])


def kernel(iteration):
    return _router_sc(iteration.astype(jnp.int32)).T


# speedup vs baseline: 1.0061x; 1.0061x over previous
import functools

import jax
import jax.numpy as jnp
from jax import lax
from jax.experimental import pallas as pl
from jax.experimental.pallas import tpu as pltpu
from jax.experimental.pallas import tpu_sc as plsc

_B = 16384
_TAU = 5
_NC = 2
_NS = 16
_NW = _NC * _NS
_L = 16
_PER_W = _B // _NW

_mesh = plsc.VectorSubcoreMesh(core_axis_name="c", subcore_axis_name="s")


@functools.partial(
    pl.kernel,
    mesh=_mesh,
    out_type=jax.ShapeDtypeStruct((2, _B), jnp.float32),
    scratch_types=[
        pltpu.VMEM((_PER_W,), jnp.int32),
        pltpu.VMEM((_PER_W,), jnp.float32),
        pltpu.VMEM((_PER_W,), jnp.float32),
        pltpu.SemaphoreType.DMA,
        pltpu.SemaphoreType.DMA,
    ],
    compiler_params=pltpu.CompilerParams(needs_layout_passes=False),
)
def _router_sc(it_hbm, out_hbm, it_v, c0_v, c1_v, sem0, sem1):
    wid = lax.axis_index("s") * _NC + lax.axis_index("c")
    base = wid * _PER_W
    pltpu.sync_copy(it_hbm.at[pl.ds(base, _PER_W)], it_v)

    def body(i, carry):
        sl = pl.ds(i * _L, _L)
        x = it_v[sl]
        hit = lax.rem(x + 1, _TAU) == 0
        col1 = jnp.where(hit, jnp.float32(1.0), jnp.float32(0.0))
        c1_v[sl] = col1
        c0_v[sl] = jnp.float32(1.0) - col1
        return carry

    lax.fori_loop(0, _PER_W // _L, body, 0)
    cp0 = pltpu.async_copy(c0_v, out_hbm.at[0, pl.ds(base, _PER_W)], sem0)
    cp1 = pltpu.async_copy(c1_v, out_hbm.at[1, pl.ds(base, _PER_W)], sem1)
    cp0.wait()
    cp1.wait()


def kernel(iteration):
    return _router_sc(iteration.astype(jnp.int32)).T


# speedup vs baseline: 1.0798x; 1.0733x over previous
import functools

import jax
import jax.numpy as jnp
from jax import lax
from jax.experimental import pallas as pl
from jax.experimental.pallas import tpu as pltpu
from jax.experimental.pallas import tpu_sc as plsc

_B = 16384
_TAU = 5
_NC = 1
_NS = 16
_NW = _NC * _NS
_L = 16
_PER_W = _B // _NW

_mesh = plsc.VectorSubcoreMesh(
    core_axis_name="c", subcore_axis_name="s", num_cores=_NC
)


@functools.partial(
    pl.kernel,
    mesh=_mesh,
    out_type=jax.ShapeDtypeStruct((2, _B), jnp.float32),
    scratch_types=[
        pltpu.VMEM((_PER_W,), jnp.int32),
        pltpu.VMEM((_PER_W,), jnp.float32),
        pltpu.VMEM((_PER_W,), jnp.float32),
        pltpu.SemaphoreType.DMA,
        pltpu.SemaphoreType.DMA,
    ],
    compiler_params=pltpu.CompilerParams(needs_layout_passes=False),
)
def _router_sc(it_hbm, out_hbm, it_v, c0_v, c1_v, sem0, sem1):
    wid = lax.axis_index("s") * _NC + lax.axis_index("c")
    base = wid * _PER_W
    pltpu.sync_copy(it_hbm.at[pl.ds(base, _PER_W)], it_v)

    def body(i, carry):
        sl = pl.ds(i * _L, _L)
        x = it_v[sl]
        hit = lax.rem(x + 1, _TAU) == 0
        col1 = jnp.where(hit, jnp.float32(1.0), jnp.float32(0.0))
        c1_v[sl] = col1
        c0_v[sl] = jnp.float32(1.0) - col1
        return carry

    lax.fori_loop(0, _PER_W // _L, body, 0)
    cp0 = pltpu.async_copy(c0_v, out_hbm.at[0, pl.ds(base, _PER_W)], sem0)
    cp1 = pltpu.async_copy(c1_v, out_hbm.at[1, pl.ds(base, _PER_W)], sem1)
    cp0.wait()
    cp1.wait()


def kernel(iteration):
    return _router_sc(iteration.astype(jnp.int32)).T
